# baseline (speedup 1.0000x reference)
import functools

import jax
import jax.numpy as jnp
from jax import lax
from jax.experimental import pallas as pl
from jax.experimental.pallas import tpu as pltpu
from jax.experimental.pallas import tpu_sc as plsc

N = 10000
E = 320000
D = 128
H1, C1 = 8, 16
F1 = H1 * C1
C2 = 40
C2P = 48
ACC1_W = F1 + 16
ACC2_W = 64

NC, NS = 2, 16
NW = NC * NS
NP = 10112
BN = 1000


def _lrelu(v):
    return jnp.where(v >= 0, v, 0.2 * v)


_SC_PARAMS = pltpu.CompilerParams(use_tc_tiling_on_sc=False,
                                  needs_layout_passes=False)


def _make_fused_edge_pass(width, acc_w, n_heads, unroll, ch, grp):
    CH = ch
    G = grp
    assert G % 2 == 0
    per_w = E // NW
    n_ch = per_w // CH
    n_grp = n_ch // G
    n_vr = width // 16
    rows_per_tile = NP // NS
    mesh = plsc.VectorSubcoreMesh(core_axis_name="c", subcore_axis_name="s")

    @functools.partial(
        pl.kernel,
        out_type=jax.ShapeDtypeStruct((2 * NP, acc_w), jnp.float32),
        mesh=mesh,
        scratch_types=[
            pltpu.VMEM((3, G, CH), jnp.int32),
            pltpu.VMEM((3, G, CH), jnp.int32),
            pltpu.VMEM((2, CH, width), jnp.float32),
            pltpu.VMEM((2, CH, width), jnp.float32),
            pltpu.VMEM((2, CH, acc_w), jnp.float32),
            pltpu.VMEM((width // 16, 16), jnp.float32),
            pltpu.VMEM_SHARED((NP, acc_w), jnp.float32),
            pltpu.SemaphoreType.DMA,
            pltpu.SemaphoreType.DMA,
            pltpu.SemaphoreType.DMA,
            pltpu.SemaphoreType.DMA,
            pltpu.SemaphoreType.DMA,
            pltpu.SemaphoreType.DMA,
            pltpu.SemaphoreType.DMA,
        ],
        compiler_params=_SC_PARAMS,
    )
    def fused_kernel(tl, tr, s3d, d3d, att, zeros, out,
                     sidx, didx, xj_v, xi_v, ov, att_v, acc,
                     gj0, gj1, gi0, gi1, ss0, ss1, slab_sem):
        gj = (gj0, gj1)
        gi = (gi0, gi1)
        ss = (ss0, ss1)
        c = lax.axis_index("c")
        s = lax.axis_index("s")
        wid = c * NS + s

        pltpu.sync_copy(att, att_v)
        r0 = s * rows_per_tile
        pltpu.sync_copy(zeros, acc.at[pl.ds(r0, rows_per_tile)])

        def start_slab(g, slot):
            pltpu.async_copy(s3d.at[wid, g], sidx.at[slot], slab_sem)
            pltpu.async_copy(d3d.at[wid, g], didx.at[slot], slab_sem)

        def wait_slab():
            pltpu.make_async_copy(s3d.at[wid, 0], sidx.at[0],
                                  slab_sem).wait()
            pltpu.make_async_copy(d3d.at[wid, 0], didx.at[0],
                                  slab_sem).wait()

        start_slab(0, 0)
        wait_slab()
        start_slab(1, 1)
        plsc.subcore_barrier()

        vr_per_h = n_vr // n_heads
        att_r = [att_v[r, :] for r in range(n_vr)]
        iota = lax.iota(jnp.int32, 16)
        onehot = [iota == h for h in range(n_heads)]
        def compute_chunk(b, slot, j):
            @plsc.parallel_loop(0, CH, 1, unroll=unroll)
            def edge_body(e):
                alpha = jnp.zeros((16,), jnp.float32)
                for h in range(n_heads):
                    acc_s = jnp.float32(0.0)
                    for v in range(vr_per_h):
                        r = h * vr_per_h + v
                        sl = pl.ds(r * 16, 16)
                        x = xi_v[b, e, sl] + xj_v[b, e, sl]
                        acc_s = acc_s + jnp.sum(
                            jnp.maximum(x, 0.2 * x) * att_r[r])
                    alpha = jnp.where(onehot[h],
                                      jax.lax.broadcast(acc_s, (16,)), alpha)
                w = jnp.exp(alpha)
                for h in range(n_heads):
                    wh = jax.lax.broadcast(w[h], (16,))
                    for v in range(vr_per_h):
                        r = h * vr_per_h + v
                        sl = pl.ds(r * 16, 16)
                        ov[b, e, sl] = xj_v[b, e, sl] * wh
                ov[b, e, pl.ds(width, 16)] = w

        def start_gathers(b, slot, j):
            pltpu.async_copy(tl.at[sidx.at[slot, j]], xj_v.at[b], gj[b])
            pltpu.async_copy(tr.at[didx.at[slot, j]], xi_v.at[b], gi[b])

        def wait_gathers(b, slot, j):
            pltpu.make_async_copy(tl.at[sidx.at[slot, j]], xj_v.at[b],
                                  gj[b]).wait()
            pltpu.make_async_copy(tr.at[didx.at[slot, j]], xi_v.at[b],
                                  gi[b]).wait()

        start_gathers(0, 0, 0)
        start_gathers(1, 0, 1)

        def group_body(g, carry):
            g3 = g % 3

            @pl.when(g + 1 < n_grp)
            def _():
                wait_slab()

            @pl.when(g + 2 < n_grp)
            def _():
                start_slab(g + 2, (g + 2) % 3)

            for j in range(G):
                b = j % 2
                k = g * G + j
                wait_gathers(b, g3, j)

                @pl.when(k >= 2)
                def _():
                    pltpu.make_async_copy(ov.at[b], acc.at[didx.at[g3, j]],
                                          ss[b]).wait()

                compute_chunk(b, g3, j)
                pltpu.async_copy(ov.at[b], acc.at[didx.at[g3, j]], ss[b],
                                 add=True)
                if j < G - 2:
                    start_gathers(b, g3, j + 2)
                else:
                    @pl.when(g + 1 < n_grp)
                    def _():
                        start_gathers(b, (g + 1) % 3, j + 2 - G)
            return carry

        lax.fori_loop(0, n_grp, group_body, 0)
        for b in range(2):
            pltpu.make_async_copy(ov.at[b], acc.at[didx.at[0, 0]],
                                  ss[b]).wait()
        plsc.subcore_barrier()
        pltpu.sync_copy(acc.at[pl.ds(r0, rows_per_tile)],
                        out.at[pl.ds(c * NP + r0, rows_per_tile)])

    return fused_kernel


def _project(x, wl, bl, wr, br):
    n, d = x.shape
    f = wl.shape[1]

    def body(x_ref, wl_ref, bl_ref, wr_ref, br_ref, xl_ref, xr_ref):
        xb = x_ref[...]
        xl_ref[...] = jnp.dot(xb, wl_ref[...],
                              preferred_element_type=jnp.float32) + bl_ref[...]
        xr_ref[...] = jnp.dot(xb, wr_ref[...],
                              preferred_element_type=jnp.float32) + br_ref[...]

    return pl.pallas_call(
        body,
        grid=(n // BN,),
        in_specs=[
            pl.BlockSpec((BN, d), lambda i: (i, 0)),
            pl.BlockSpec((d, f), lambda i: (0, 0)),
            pl.BlockSpec((1, f), lambda i: (0, 0)),
            pl.BlockSpec((d, f), lambda i: (0, 0)),
            pl.BlockSpec((1, f), lambda i: (0, 0)),
        ],
        out_specs=[pl.BlockSpec((BN, f), lambda i: (i, 0))] * 2,
        out_shape=[jax.ShapeDtypeStruct((n, f), jnp.float32)] * 2,
    )(x, wl, bl.reshape(1, -1), wr, br.reshape(1, -1))


def _mid_stage(acc0, acc1, xl1, xr1, a_mat, b_mat, bias1,
               wl2p, bl2p, wr2p, br2p, att2p):
    def body(a0_ref, a1_ref, xl_ref, xr_ref, a_ref, b_ref, b1_ref,
             wl2_ref, bl2_ref, wr2_ref, br2_ref, att2_ref,
             xl2_ref, xr2_ref, s2_ref):
        xl_ = xl_ref[...]
        xr_ = xr_ref[...]
        l = _lrelu(xl_ + xr_)
        wii = jnp.exp(jnp.dot(l, a_ref[...],
                              preferred_element_type=jnp.float32))
        t0 = a0_ref[...]
        t1 = a1_ref[...]
        num = (t0[:, :F1] + t1[:, :F1]
               + jnp.dot(wii, b_ref[...],
                         preferred_element_type=jnp.float32) * xl_)
        den = t0[:, F1:F1 + H1] + t1[:, F1:F1 + H1] + wii
        inv = 1.0 / (den + 1e-16)
        hin = num * jnp.dot(inv, b_ref[...],
                            preferred_element_type=jnp.float32) + b1_ref[...]
        h = jnp.where(hin > 0, hin, jnp.exp(jnp.minimum(hin, 0.0)) - 1.0)
        xl2 = jnp.dot(h, wl2_ref[...],
                      preferred_element_type=jnp.float32) + bl2_ref[...]
        xr2 = jnp.dot(h, wr2_ref[...],
                      preferred_element_type=jnp.float32) + br2_ref[...]
        l2 = _lrelu(xl2 + xr2)
        w2 = jnp.exp(jnp.dot(l2, att2_ref[...],
                             preferred_element_type=jnp.float32))
        z = jnp.zeros((xl2.shape[0], ACC2_W - C2P - 1), jnp.float32)
        xl2_ref[...] = xl2
        xr2_ref[...] = xr2
        s2_ref[...] = jnp.concatenate([w2 * xl2, w2, z], axis=1)

    return pl.pallas_call(
        body,
        grid=(N // BN,),
        in_specs=[
            pl.BlockSpec((BN, ACC1_W), lambda i: (i, 0)),
            pl.BlockSpec((BN, ACC1_W), lambda i: (i, 0)),
            pl.BlockSpec((BN, F1), lambda i: (i, 0)),
            pl.BlockSpec((BN, F1), lambda i: (i, 0)),
            pl.BlockSpec((F1, H1), lambda i: (0, 0)),
            pl.BlockSpec((H1, F1), lambda i: (0, 0)),
            pl.BlockSpec((1, F1), lambda i: (0, 0)),
            pl.BlockSpec((F1, C2P), lambda i: (0, 0)),
            pl.BlockSpec((1, C2P), lambda i: (0, 0)),
            pl.BlockSpec((F1, C2P), lambda i: (0, 0)),
            pl.BlockSpec((1, C2P), lambda i: (0, 0)),
            pl.BlockSpec((C2P, 1), lambda i: (0, 0)),
        ],
        out_specs=[
            pl.BlockSpec((BN, C2P), lambda i: (i, 0)),
            pl.BlockSpec((BN, C2P), lambda i: (i, 0)),
            pl.BlockSpec((BN, ACC2_W), lambda i: (i, 0)),
        ],
        out_shape=[
            jax.ShapeDtypeStruct((N, C2P), jnp.float32),
            jax.ShapeDtypeStruct((N, C2P), jnp.float32),
            jax.ShapeDtypeStruct((N, ACC2_W), jnp.float32),
        ],
    )(acc0, acc1, xl1, xr1, a_mat, b_mat, bias1,
      wl2p, bl2p, wr2p, br2p, att2p)


def _final_stage(acc0, acc1, s2, bias2):
    def body(a0_ref, a1_ref, s2_ref, b2_ref, out_ref):
        t = a0_ref[...] + a1_ref[...] + s2_ref[...]
        den = t[:, C2P:C2P + 1]
        out_ref[...] = t[:, :C2] / (den + 1e-16) + b2_ref[...]

    return pl.pallas_call(
        body,
        grid=(N // BN,),
        in_specs=[
            pl.BlockSpec((BN, ACC2_W), lambda i: (i, 0)),
            pl.BlockSpec((BN, ACC2_W), lambda i: (i, 0)),
            pl.BlockSpec((BN, ACC2_W), lambda i: (i, 0)),
            pl.BlockSpec((1, C2), lambda i: (0, 0)),
        ],
        out_specs=pl.BlockSpec((BN, C2), lambda i: (i, 0)),
        out_shape=jax.ShapeDtypeStruct((N, C2), jnp.float32),
    )(acc0, acc1, s2, bias2.reshape(1, -1))


CH1, G1 = 25, 8
CH2, G2 = 40, 10
_edge_pass1 = _make_fused_edge_pass(F1, ACC1_W, H1, unroll=4, ch=CH1, grp=G1)
_edge_pass2 = _make_fused_edge_pass(C2P, ACC2_W, 1, unroll=4, ch=CH2, grp=G2)


def kernel(x, edge_index, Wl1, bl1, Wr1, br1, att1, bias1,
           Wl2, bl2, Wr2, br2, att2, bias2):
    s3d1 = edge_index[0].reshape(NW, E // (NW * G1 * CH1), G1, CH1)
    d3d1 = edge_index[1].reshape(NW, E // (NW * G1 * CH1), G1, CH1)
    s3d2 = edge_index[0].reshape(NW, E // (NW * G2 * CH2), G2, CH2)
    d3d2 = edge_index[1].reshape(NW, E // (NW * G2 * CH2), G2, CH2)

    a1 = (att1[:, :, None] * jnp.eye(H1, dtype=jnp.float32)[:, None, :]
          ).reshape(F1, H1)
    bm = jnp.repeat(jnp.eye(H1, dtype=jnp.float32), C1, axis=1)
    att2p = jnp.zeros((C2P, 1), jnp.float32).at[:C2, 0].set(att2[0])
    att2v = att2p.reshape(3, 16)
    wl2p = jnp.pad(Wl2, ((0, 0), (0, C2P - C2)))
    wr2p = jnp.pad(Wr2, ((0, 0), (0, C2P - C2)))
    bl2p = jnp.pad(bl2, (0, C2P - C2)).reshape(1, -1)
    br2p = jnp.pad(br2, (0, C2P - C2)).reshape(1, -1)
    zeros1 = jnp.zeros((NP // NS, ACC1_W), jnp.float32)
    zeros2 = jnp.zeros((NP // NS, ACC2_W), jnp.float32)

    xl1, xr1 = _project(x, Wl1, bl1, Wr1, br1)
    acc1 = _edge_pass1(xl1, xr1, s3d1, d3d1, att1, zeros1)
    xl2, xr2, s2 = _mid_stage(acc1[:N], acc1[NP:NP + N], xl1, xr1, a1, bm,
                              bias1.reshape(1, -1), wl2p, bl2p, wr2p, br2p,
                              att2p)
    acc2 = _edge_pass2(xl2, xr2, s3d2, d3d2, att2v, zeros2)
    return _final_stage(acc2[:N], acc2[NP:NP + N], s2, bias2)


# speedup vs baseline: 1.3953x; 1.3953x over previous
import functools

import jax
import jax.numpy as jnp
from jax import lax
from jax.experimental import pallas as pl
from jax.experimental.pallas import tpu as pltpu
from jax.experimental.pallas import tpu_sc as plsc

N = 10000
E = 320000
D = 128
H1, C1 = 8, 16
F1 = H1 * C1
C2 = 40
C2P = 48
ACC1_W = F1 + 16
ACC2_W = 64

NC, NS = 2, 16
NW = NC * NS
NP = 10112
BN = 1000


def _lrelu(v):
    return jnp.where(v >= 0, v, 0.2 * v)


_SC_PARAMS = pltpu.CompilerParams(use_tc_tiling_on_sc=False,
                                  needs_layout_passes=False)


def _make_fused_edge_pass(width, acc_w, n_heads, unroll, ch, grp):
    CH = ch
    G = grp
    assert G % 2 == 0
    per_w = E // NW
    n_ch = per_w // CH
    n_grp = n_ch // G
    n_vr = width // 16
    rows_per_tile = NP // NS
    mesh = plsc.VectorSubcoreMesh(core_axis_name="c", subcore_axis_name="s")

    @functools.partial(
        pl.kernel,
        out_type=jax.ShapeDtypeStruct((2 * NP, acc_w), jnp.float32),
        mesh=mesh,
        scratch_types=[
            pltpu.VMEM((3, G, CH), jnp.int32),
            pltpu.VMEM((3, G, CH), jnp.int32),
            pltpu.VMEM((2, CH, width), jnp.float32),
            pltpu.VMEM((2, CH, width), jnp.float32),
            pltpu.VMEM((2, CH, acc_w), jnp.float32),
            pltpu.VMEM((width // 16, 16), jnp.float32),
            pltpu.VMEM_SHARED((NP, acc_w), jnp.float32),
            pltpu.SemaphoreType.DMA,
            pltpu.SemaphoreType.DMA,
            pltpu.SemaphoreType.DMA,
            pltpu.SemaphoreType.DMA,
            pltpu.SemaphoreType.DMA,
            pltpu.SemaphoreType.DMA,
            pltpu.SemaphoreType.DMA,
        ],
        compiler_params=_SC_PARAMS,
    )
    def fused_kernel(tl, tr, s3d, d3d, att, zeros, out,
                     sidx, didx, xj_v, xi_v, ov, att_v, acc,
                     gj0, gj1, gi0, gi1, ss0, ss1, slab_sem):
        gj = (gj0, gj1)
        gi = (gi0, gi1)
        ss = (ss0, ss1)
        c = lax.axis_index("c")
        s = lax.axis_index("s")
        wid = c * NS + s

        pltpu.sync_copy(att, att_v)
        r0 = s * rows_per_tile
        pltpu.sync_copy(zeros, acc.at[pl.ds(r0, rows_per_tile)])

        def start_slab(g, slot):
            pltpu.async_copy(s3d.at[wid, g], sidx.at[slot], slab_sem)
            pltpu.async_copy(d3d.at[wid, g], didx.at[slot], slab_sem)

        def wait_slab():
            pltpu.make_async_copy(s3d.at[wid, 0], sidx.at[0],
                                  slab_sem).wait()
            pltpu.make_async_copy(d3d.at[wid, 0], didx.at[0],
                                  slab_sem).wait()

        start_slab(0, 0)
        wait_slab()
        start_slab(1, 1)
        plsc.subcore_barrier()

        vr_per_h = n_vr // n_heads
        att_r = [att_v[r, :] for r in range(n_vr)]
        iota = lax.iota(jnp.int32, 16)
        onehot = [iota == h for h in range(n_heads)]
        def compute_chunk(b, slot, j):
            @plsc.parallel_loop(0, CH, 1, unroll=unroll)
            def edge_body(e):
                alpha = jnp.zeros((16,), jnp.float32)
                for h in range(n_heads):
                    acc_s = jnp.float32(0.0)
                    for v in range(vr_per_h):
                        r = h * vr_per_h + v
                        sl = pl.ds(r * 16, 16)
                        x = xi_v[b, e, sl] + xj_v[b, e, sl]
                        acc_s = acc_s + jnp.sum(
                            jnp.maximum(x, 0.2 * x) * att_r[r])
                    alpha = jnp.where(onehot[h],
                                      jax.lax.broadcast(acc_s, (16,)), alpha)
                w = jnp.exp(alpha)
                for h in range(n_heads):
                    wh = jax.lax.broadcast(w[h], (16,))
                    for v in range(vr_per_h):
                        r = h * vr_per_h + v
                        sl = pl.ds(r * 16, 16)
                        ov[b, e, sl] = xj_v[b, e, sl] * wh
                ov[b, e, pl.ds(width, 16)] = w

        def start_gathers(b, slot, j):
            pltpu.async_copy(tl.at[sidx.at[slot, j]], xj_v.at[b], gj[b])
            pltpu.async_copy(tr.at[didx.at[slot, j]], xi_v.at[b], gi[b])

        def wait_gathers(b, slot, j):
            pltpu.make_async_copy(tl.at[sidx.at[slot, j]], xj_v.at[b],
                                  gj[b]).wait()
            pltpu.make_async_copy(tr.at[didx.at[slot, j]], xi_v.at[b],
                                  gi[b]).wait()

        start_gathers(0, 0, 0)
        start_gathers(1, 0, 1)

        def group_body(g, carry):
            g3 = g % 3

            @pl.when(g + 1 < n_grp)
            def _():
                wait_slab()

            @pl.when(g + 2 < n_grp)
            def _():
                start_slab(g + 2, (g + 2) % 3)

            for j in range(G):
                b = j % 2
                k = g * G + j
                wait_gathers(b, g3, j)

                @pl.when(k >= 2)
                def _():
                    pltpu.make_async_copy(ov.at[b], acc.at[didx.at[g3, j]],
                                          ss[b]).wait()

                compute_chunk(b, g3, j)
                pltpu.async_copy(ov.at[b], acc.at[didx.at[g3, j]], ss[b],
                                 add=True)
                if j < G - 2:
                    start_gathers(b, g3, j + 2)
                else:
                    @pl.when(g + 1 < n_grp)
                    def _():
                        start_gathers(b, (g + 1) % 3, j + 2 - G)
            return carry

        lax.fori_loop(0, n_grp, group_body, 0)
        for b in range(2):
            pltpu.make_async_copy(ov.at[b], acc.at[didx.at[0, 0]],
                                  ss[b]).wait()
        plsc.subcore_barrier()
        pltpu.sync_copy(acc.at[pl.ds(r0, rows_per_tile)],
                        out.at[pl.ds(c * NP + r0, rows_per_tile)])

    return fused_kernel


def _project(x, wl, bl, wr, br):
    n, d = x.shape
    f = wl.shape[1]

    def body(x_ref, wl_ref, bl_ref, wr_ref, br_ref, xl_ref, xr_ref):
        xb = x_ref[...]
        xl_ref[...] = jnp.dot(xb, wl_ref[...],
                              preferred_element_type=jnp.float32) + bl_ref[...]
        xr_ref[...] = jnp.dot(xb, wr_ref[...],
                              preferred_element_type=jnp.float32) + br_ref[...]

    return pl.pallas_call(
        body,
        grid=(n // BN,),
        in_specs=[
            pl.BlockSpec((BN, d), lambda i: (i, 0)),
            pl.BlockSpec((d, f), lambda i: (0, 0)),
            pl.BlockSpec((1, f), lambda i: (0, 0)),
            pl.BlockSpec((d, f), lambda i: (0, 0)),
            pl.BlockSpec((1, f), lambda i: (0, 0)),
        ],
        out_specs=[pl.BlockSpec((BN, f), lambda i: (i, 0))] * 2,
        out_shape=[jax.ShapeDtypeStruct((n, f), jnp.float32)] * 2,
    )(x, wl, bl.reshape(1, -1), wr, br.reshape(1, -1))


def _mid_stage(acc0, acc1, xl1, xr1, a_mat, b_mat, bias1,
               wl2p, bl2p, wr2p, br2p, att2p):
    def body(a0_ref, a1_ref, xl_ref, xr_ref, a_ref, b_ref, b1_ref,
             wl2_ref, bl2_ref, wr2_ref, br2_ref, att2_ref,
             xl2_ref, xr2_ref, s2_ref):
        xl_ = xl_ref[...]
        xr_ = xr_ref[...]
        l = _lrelu(xl_ + xr_)
        wii = jnp.exp(jnp.dot(l, a_ref[...],
                              preferred_element_type=jnp.float32))
        t0 = a0_ref[...]
        t1 = a1_ref[...]
        num = (t0[:, :F1] + t1[:, :F1]
               + jnp.dot(wii, b_ref[...],
                         preferred_element_type=jnp.float32) * xl_)
        den = t0[:, F1:F1 + H1] + t1[:, F1:F1 + H1] + wii
        inv = 1.0 / (den + 1e-16)
        hin = num * jnp.dot(inv, b_ref[...],
                            preferred_element_type=jnp.float32) + b1_ref[...]
        h = jnp.where(hin > 0, hin, jnp.exp(jnp.minimum(hin, 0.0)) - 1.0)
        xl2 = jnp.dot(h, wl2_ref[...],
                      preferred_element_type=jnp.float32) + bl2_ref[...]
        xr2 = jnp.dot(h, wr2_ref[...],
                      preferred_element_type=jnp.float32) + br2_ref[...]
        l2 = _lrelu(xl2 + xr2)
        w2 = jnp.exp(jnp.dot(l2, att2_ref[...],
                             preferred_element_type=jnp.float32))
        z = jnp.zeros((xl2.shape[0], ACC2_W - C2P - 1), jnp.float32)
        xl2_ref[...] = xl2
        xr2_ref[...] = xr2
        s2_ref[...] = jnp.concatenate([w2 * xl2, w2, z], axis=1)

    return pl.pallas_call(
        body,
        grid=(N // BN,),
        in_specs=[
            pl.BlockSpec((BN, ACC1_W), lambda i: (i, 0)),
            pl.BlockSpec((BN, ACC1_W), lambda i: (i, 0)),
            pl.BlockSpec((BN, F1), lambda i: (i, 0)),
            pl.BlockSpec((BN, F1), lambda i: (i, 0)),
            pl.BlockSpec((F1, H1), lambda i: (0, 0)),
            pl.BlockSpec((H1, F1), lambda i: (0, 0)),
            pl.BlockSpec((1, F1), lambda i: (0, 0)),
            pl.BlockSpec((F1, C2P), lambda i: (0, 0)),
            pl.BlockSpec((1, C2P), lambda i: (0, 0)),
            pl.BlockSpec((F1, C2P), lambda i: (0, 0)),
            pl.BlockSpec((1, C2P), lambda i: (0, 0)),
            pl.BlockSpec((C2P, 1), lambda i: (0, 0)),
        ],
        out_specs=[
            pl.BlockSpec((BN, C2P), lambda i: (i, 0)),
            pl.BlockSpec((BN, C2P), lambda i: (i, 0)),
            pl.BlockSpec((BN, ACC2_W), lambda i: (i, 0)),
        ],
        out_shape=[
            jax.ShapeDtypeStruct((N, C2P), jnp.float32),
            jax.ShapeDtypeStruct((N, C2P), jnp.float32),
            jax.ShapeDtypeStruct((N, ACC2_W), jnp.float32),
        ],
    )(acc0, acc1, xl1, xr1, a_mat, b_mat, bias1,
      wl2p, bl2p, wr2p, br2p, att2p)


def _final_stage(acc0, acc1, s2, bias2):
    def body(a0_ref, a1_ref, s2_ref, b2_ref, out_ref):
        t = a0_ref[...] + a1_ref[...] + s2_ref[...]
        den = t[:, C2P:C2P + 1]
        out_ref[...] = t[:, :C2] / (den + 1e-16) + b2_ref[...]

    return pl.pallas_call(
        body,
        grid=(N // BN,),
        in_specs=[
            pl.BlockSpec((BN, ACC2_W), lambda i: (i, 0)),
            pl.BlockSpec((BN, ACC2_W), lambda i: (i, 0)),
            pl.BlockSpec((BN, ACC2_W), lambda i: (i, 0)),
            pl.BlockSpec((1, C2), lambda i: (0, 0)),
        ],
        out_specs=pl.BlockSpec((BN, C2), lambda i: (i, 0)),
        out_shape=jax.ShapeDtypeStruct((N, C2), jnp.float32),
    )(acc0, acc1, s2, bias2.reshape(1, -1))


CH1, G1 = 40, 10
CH2, G2 = 40, 10
_edge_pass1 = _make_fused_edge_pass(F1, ACC1_W, H1, unroll=2, ch=CH1, grp=G1)
_edge_pass2 = _make_fused_edge_pass(C2P, ACC2_W, 1, unroll=8, ch=CH2, grp=G2)


def kernel(x, edge_index, Wl1, bl1, Wr1, br1, att1, bias1,
           Wl2, bl2, Wr2, br2, att2, bias2):
    s3d1 = edge_index[0].reshape(NW, E // (NW * G1 * CH1), G1, CH1)
    d3d1 = edge_index[1].reshape(NW, E // (NW * G1 * CH1), G1, CH1)
    s3d2 = edge_index[0].reshape(NW, E // (NW * G2 * CH2), G2, CH2)
    d3d2 = edge_index[1].reshape(NW, E // (NW * G2 * CH2), G2, CH2)

    a1 = (att1[:, :, None] * jnp.eye(H1, dtype=jnp.float32)[:, None, :]
          ).reshape(F1, H1)
    bm = jnp.repeat(jnp.eye(H1, dtype=jnp.float32), C1, axis=1)
    att2p = jnp.zeros((C2P, 1), jnp.float32).at[:C2, 0].set(att2[0])
    att2v = att2p.reshape(3, 16)
    wl2p = jnp.pad(Wl2, ((0, 0), (0, C2P - C2)))
    wr2p = jnp.pad(Wr2, ((0, 0), (0, C2P - C2)))
    bl2p = jnp.pad(bl2, (0, C2P - C2)).reshape(1, -1)
    br2p = jnp.pad(br2, (0, C2P - C2)).reshape(1, -1)
    zeros1 = jnp.zeros((NP // NS, ACC1_W), jnp.float32)
    zeros2 = jnp.zeros((NP // NS, ACC2_W), jnp.float32)

    xl1, xr1 = _project(x, Wl1, bl1, Wr1, br1)
    acc1 = _edge_pass1(xl1, xr1, s3d1, d3d1, att1, zeros1)
    xl2, xr2, s2 = _mid_stage(acc1[:N], acc1[NP:NP + N], xl1, xr1, a1, bm,
                              bias1.reshape(1, -1), wl2p, bl2p, wr2p, br2p,
                              att2p)
    acc2 = _edge_pass2(xl2, xr2, s3d2, d3d2, att2v, zeros2)
    return _final_stage(acc2[:N], acc2[NP:NP + N], s2, bias2)


# speedup vs baseline: 1.5406x; 1.1041x over previous
import functools

import jax
import jax.numpy as jnp
from jax import lax
from jax.experimental import pallas as pl
from jax.experimental.pallas import tpu as pltpu
from jax.experimental.pallas import tpu_sc as plsc

N = 10000
E = 320000
D = 128
H1, C1 = 8, 16
F1 = H1 * C1
C2 = 40
C2P = 48
ACC1_W = F1 + 16
ACC2_W = 64

NC, NS = 2, 16
NW = NC * NS
NP = 10112
BN = 1000


def _lrelu(v):
    return jnp.where(v >= 0, v, 0.2 * v)


_SC_PARAMS = pltpu.CompilerParams(use_tc_tiling_on_sc=False,
                                  needs_layout_passes=False)


def _make_fused_edge_pass(width, acc_w, n_heads, unroll, ch, grp):
    CH = ch
    G = grp
    assert G % 2 == 0
    per_w = E // NW
    n_ch = per_w // CH
    n_grp = n_ch // G
    n_vr = width // 16
    rows_per_tile = NP // NS
    mesh = plsc.VectorSubcoreMesh(core_axis_name="c", subcore_axis_name="s")

    @functools.partial(
        pl.kernel,
        out_type=jax.ShapeDtypeStruct((2 * NP, acc_w), jnp.float32),
        mesh=mesh,
        scratch_types=[
            pltpu.VMEM((3, G, CH), jnp.int32),
            pltpu.VMEM((3, G, CH), jnp.int32),
            pltpu.VMEM((2, CH, width), jnp.float32),
            pltpu.VMEM((2, CH, width), jnp.float32),
            pltpu.VMEM((2, CH, acc_w), jnp.float32),
            pltpu.VMEM((width // 16, 16), jnp.float32),
            pltpu.VMEM_SHARED((NP, acc_w), jnp.float32),
            pltpu.SemaphoreType.DMA,
            pltpu.SemaphoreType.DMA,
            pltpu.SemaphoreType.DMA,
            pltpu.SemaphoreType.DMA,
            pltpu.SemaphoreType.DMA,
            pltpu.SemaphoreType.DMA,
            pltpu.SemaphoreType.DMA,
        ],
        compiler_params=_SC_PARAMS,
    )
    def fused_kernel(tl, tr, s3d, d3d, att, zeros, out,
                     sidx, didx, xj_v, xi_v, ov, att_v, acc,
                     gj0, gj1, gi0, gi1, ss0, ss1, slab_sem):
        gj = (gj0, gj1)
        gi = (gi0, gi1)
        ss = (ss0, ss1)
        c = lax.axis_index("c")
        s = lax.axis_index("s")
        wid = c * NS + s

        pltpu.sync_copy(att, att_v)
        r0 = s * rows_per_tile
        pltpu.sync_copy(zeros, acc.at[pl.ds(r0, rows_per_tile)])

        def start_slab(g, slot):
            pltpu.async_copy(s3d.at[wid, g], sidx.at[slot], slab_sem)
            pltpu.async_copy(d3d.at[wid, g], didx.at[slot], slab_sem)

        def wait_slab():
            pltpu.make_async_copy(s3d.at[wid, 0], sidx.at[0],
                                  slab_sem).wait()
            pltpu.make_async_copy(d3d.at[wid, 0], didx.at[0],
                                  slab_sem).wait()

        start_slab(0, 0)
        wait_slab()
        start_slab(1, 1)
        plsc.subcore_barrier()

        vr_per_h = n_vr // n_heads
        att_r = [att_v[r, :] for r in range(n_vr)]
        iota = lax.iota(jnp.int32, 16)
        onehot = [iota == h for h in range(n_heads)]
        def compute_chunk(b, slot, j):
            @plsc.parallel_loop(0, CH, 1, unroll=unroll)
            def edge_body(e):
                alpha = jnp.zeros((16,), jnp.float32)
                for h in range(n_heads):
                    acc_s = jnp.float32(0.0)
                    for v in range(vr_per_h):
                        r = h * vr_per_h + v
                        sl = pl.ds(r * 16, 16)
                        x = xi_v[b, e, sl] + xj_v[b, e, sl]
                        acc_s = acc_s + jnp.sum(
                            jnp.maximum(x, 0.2 * x) * att_r[r])
                    alpha = jnp.where(onehot[h],
                                      jax.lax.broadcast(acc_s, (16,)), alpha)
                w = jnp.exp(alpha)
                for h in range(n_heads):
                    wh = jax.lax.broadcast(w[h], (16,))
                    for v in range(vr_per_h):
                        r = h * vr_per_h + v
                        sl = pl.ds(r * 16, 16)
                        ov[b, e, sl] = xj_v[b, e, sl] * wh
                ov[b, e, pl.ds(width, 16)] = w

        def start_gathers(b, slot, j):
            pltpu.async_copy(tl.at[sidx.at[slot, j]], xj_v.at[b], gj[b])
            pltpu.async_copy(tr.at[didx.at[slot, j]], xi_v.at[b], gi[b])

        def wait_gathers(b, slot, j):
            pltpu.make_async_copy(tl.at[sidx.at[slot, j]], xj_v.at[b],
                                  gj[b]).wait()
            pltpu.make_async_copy(tr.at[didx.at[slot, j]], xi_v.at[b],
                                  gi[b]).wait()

        start_gathers(0, 0, 0)
        start_gathers(1, 0, 1)

        def group_body(g, carry):
            g3 = g % 3

            @pl.when(g + 1 < n_grp)
            def _():
                wait_slab()

            @pl.when(g + 2 < n_grp)
            def _():
                start_slab(g + 2, (g + 2) % 3)

            for j in range(G):
                b = j % 2
                k = g * G + j
                wait_gathers(b, g3, j)

                @pl.when(k >= 2)
                def _():
                    pltpu.make_async_copy(ov.at[b], acc.at[didx.at[g3, j]],
                                          ss[b]).wait()

                compute_chunk(b, g3, j)
                pltpu.async_copy(ov.at[b], acc.at[didx.at[g3, j]], ss[b],
                                 add=True)
                if j < G - 2:
                    start_gathers(b, g3, j + 2)
                else:
                    @pl.when(g + 1 < n_grp)
                    def _():
                        start_gathers(b, (g + 1) % 3, j + 2 - G)
            return carry

        lax.fori_loop(0, n_grp, group_body, 0)
        for b in range(2):
            pltpu.make_async_copy(ov.at[b], acc.at[didx.at[0, 0]],
                                  ss[b]).wait()
        plsc.subcore_barrier()
        pltpu.sync_copy(acc.at[pl.ds(r0, rows_per_tile)],
                        out.at[pl.ds(c * NP + r0, rows_per_tile)])

    return fused_kernel


def _project(x, wl, bl, wr, br):
    n, d = x.shape
    f = wl.shape[1]

    def body(x_ref, wl_ref, bl_ref, wr_ref, br_ref, xl_ref, xr_ref):
        xb = x_ref[...]
        xl_ref[...] = jnp.dot(xb, wl_ref[...],
                              preferred_element_type=jnp.float32) + bl_ref[...]
        xr_ref[...] = jnp.dot(xb, wr_ref[...],
                              preferred_element_type=jnp.float32) + br_ref[...]

    return pl.pallas_call(
        body,
        grid=(n // BN,),
        in_specs=[
            pl.BlockSpec((BN, d), lambda i: (i, 0)),
            pl.BlockSpec((d, f), lambda i: (0, 0)),
            pl.BlockSpec((1, f), lambda i: (0, 0)),
            pl.BlockSpec((d, f), lambda i: (0, 0)),
            pl.BlockSpec((1, f), lambda i: (0, 0)),
        ],
        out_specs=[pl.BlockSpec((BN, f), lambda i: (i, 0))] * 2,
        out_shape=[jax.ShapeDtypeStruct((n, f), jnp.float32)] * 2,
    )(x, wl, bl.reshape(1, -1), wr, br.reshape(1, -1))


def _mid_stage(acc0, acc1, xl1, xr1, a_mat, b_mat, bias1,
               wl2p, bl2p, wr2p, br2p, att2p):
    def body(a0_ref, a1_ref, xl_ref, xr_ref, a_ref, b_ref, b1_ref,
             wl2_ref, bl2_ref, wr2_ref, br2_ref, att2_ref,
             xl2_ref, xr2_ref, s2_ref):
        xl_ = xl_ref[...]
        xr_ = xr_ref[...]
        l = _lrelu(xl_ + xr_)
        wii = jnp.exp(jnp.dot(l, a_ref[...],
                              preferred_element_type=jnp.float32))
        t0 = a0_ref[...]
        t1 = a1_ref[...]
        num = (t0[:, :F1] + t1[:, :F1]
               + jnp.dot(wii, b_ref[...],
                         preferred_element_type=jnp.float32) * xl_)
        den = t0[:, F1:F1 + H1] + t1[:, F1:F1 + H1] + wii
        inv = 1.0 / (den + 1e-16)
        hin = num * jnp.dot(inv, b_ref[...],
                            preferred_element_type=jnp.float32) + b1_ref[...]
        h = jnp.where(hin > 0, hin, jnp.exp(jnp.minimum(hin, 0.0)) - 1.0)
        xl2 = jnp.dot(h, wl2_ref[...],
                      preferred_element_type=jnp.float32) + bl2_ref[...]
        xr2 = jnp.dot(h, wr2_ref[...],
                      preferred_element_type=jnp.float32) + br2_ref[...]
        l2 = _lrelu(xl2 + xr2)
        w2 = jnp.exp(jnp.dot(l2, att2_ref[...],
                             preferred_element_type=jnp.float32))
        z = jnp.zeros((xl2.shape[0], ACC2_W - C2P - 1), jnp.float32)
        xl2_ref[...] = xl2
        xr2_ref[...] = xr2
        s2_ref[...] = jnp.concatenate([w2 * xl2, w2, z], axis=1)

    return pl.pallas_call(
        body,
        grid=(N // BN,),
        in_specs=[
            pl.BlockSpec((BN, ACC1_W), lambda i: (i, 0)),
            pl.BlockSpec((BN, ACC1_W), lambda i: (i, 0)),
            pl.BlockSpec((BN, F1), lambda i: (i, 0)),
            pl.BlockSpec((BN, F1), lambda i: (i, 0)),
            pl.BlockSpec((F1, H1), lambda i: (0, 0)),
            pl.BlockSpec((H1, F1), lambda i: (0, 0)),
            pl.BlockSpec((1, F1), lambda i: (0, 0)),
            pl.BlockSpec((F1, C2P), lambda i: (0, 0)),
            pl.BlockSpec((1, C2P), lambda i: (0, 0)),
            pl.BlockSpec((F1, C2P), lambda i: (0, 0)),
            pl.BlockSpec((1, C2P), lambda i: (0, 0)),
            pl.BlockSpec((C2P, 1), lambda i: (0, 0)),
        ],
        out_specs=[
            pl.BlockSpec((BN, C2P), lambda i: (i, 0)),
            pl.BlockSpec((BN, C2P), lambda i: (i, 0)),
            pl.BlockSpec((BN, ACC2_W), lambda i: (i, 0)),
        ],
        out_shape=[
            jax.ShapeDtypeStruct((N, C2P), jnp.float32),
            jax.ShapeDtypeStruct((N, C2P), jnp.float32),
            jax.ShapeDtypeStruct((N, ACC2_W), jnp.float32),
        ],
    )(acc0, acc1, xl1, xr1, a_mat, b_mat, bias1,
      wl2p, bl2p, wr2p, br2p, att2p)


def _final_stage(acc0, acc1, s2, bias2):
    def body(a0_ref, a1_ref, s2_ref, b2_ref, out_ref):
        t = a0_ref[...] + a1_ref[...] + s2_ref[...]
        den = t[:, C2P:C2P + 1]
        out_ref[...] = t[:, :C2] / (den + 1e-16) + b2_ref[...]

    return pl.pallas_call(
        body,
        grid=(N // BN,),
        in_specs=[
            pl.BlockSpec((BN, ACC2_W), lambda i: (i, 0)),
            pl.BlockSpec((BN, ACC2_W), lambda i: (i, 0)),
            pl.BlockSpec((BN, ACC2_W), lambda i: (i, 0)),
            pl.BlockSpec((1, C2), lambda i: (0, 0)),
        ],
        out_specs=pl.BlockSpec((BN, C2), lambda i: (i, 0)),
        out_shape=jax.ShapeDtypeStruct((N, C2), jnp.float32),
    )(acc0, acc1, s2, bias2.reshape(1, -1))


CH1, G1 = 40, 10
CH2, G2 = 100, 10
_edge_pass1 = _make_fused_edge_pass(F1, ACC1_W, H1, unroll=2, ch=CH1, grp=G1)
_edge_pass2 = _make_fused_edge_pass(C2P, ACC2_W, 1, unroll=8, ch=CH2, grp=G2)


def kernel(x, edge_index, Wl1, bl1, Wr1, br1, att1, bias1,
           Wl2, bl2, Wr2, br2, att2, bias2):
    s3d1 = edge_index[0].reshape(NW, E // (NW * G1 * CH1), G1, CH1)
    d3d1 = edge_index[1].reshape(NW, E // (NW * G1 * CH1), G1, CH1)
    s3d2 = edge_index[0].reshape(NW, E // (NW * G2 * CH2), G2, CH2)
    d3d2 = edge_index[1].reshape(NW, E // (NW * G2 * CH2), G2, CH2)

    a1 = (att1[:, :, None] * jnp.eye(H1, dtype=jnp.float32)[:, None, :]
          ).reshape(F1, H1)
    bm = jnp.repeat(jnp.eye(H1, dtype=jnp.float32), C1, axis=1)
    att2p = jnp.zeros((C2P, 1), jnp.float32).at[:C2, 0].set(att2[0])
    att2v = att2p.reshape(3, 16)
    wl2p = jnp.pad(Wl2, ((0, 0), (0, C2P - C2)))
    wr2p = jnp.pad(Wr2, ((0, 0), (0, C2P - C2)))
    bl2p = jnp.pad(bl2, (0, C2P - C2)).reshape(1, -1)
    br2p = jnp.pad(br2, (0, C2P - C2)).reshape(1, -1)
    zeros1 = jnp.zeros((NP // NS, ACC1_W), jnp.float32)
    zeros2 = jnp.zeros((NP // NS, ACC2_W), jnp.float32)

    xl1, xr1 = _project(x, Wl1, bl1, Wr1, br1)
    acc1 = _edge_pass1(xl1, xr1, s3d1, d3d1, att1, zeros1)
    xl2, xr2, s2 = _mid_stage(acc1[:N], acc1[NP:NP + N], xl1, xr1, a1, bm,
                              bias1.reshape(1, -1), wl2p, bl2p, wr2p, br2p,
                              att2p)
    acc2 = _edge_pass2(xl2, xr2, s3d2, d3d2, att2v, zeros2)
    return _final_stage(acc2[:N], acc2[NP:NP + N], s2, bias2)


# speedup vs baseline: 1.5554x; 1.0096x over previous
import functools

import jax
import jax.numpy as jnp
from jax import lax
from jax.experimental import pallas as pl
from jax.experimental.pallas import tpu as pltpu
from jax.experimental.pallas import tpu_sc as plsc

N = 10000
E = 320000
D = 128
H1, C1 = 8, 16
F1 = H1 * C1
C2 = 40
C2P = 48
ACC1_W = F1 + 16
ACC2_W = 64

NC, NS = 2, 16
NW = NC * NS
NP = 10112
BN = 632


def _lrelu(v):
    return jnp.where(v >= 0, v, 0.2 * v)


_SC_PARAMS = pltpu.CompilerParams(use_tc_tiling_on_sc=False,
                                  needs_layout_passes=False)


def _make_fused_edge_pass(width, acc_w, n_heads, unroll, ch, grp):
    CH = ch
    G = grp
    assert G % 2 == 0
    per_w = E // NW
    n_ch = per_w // CH
    n_grp = n_ch // G
    n_vr = width // 16
    rows_per_tile = NP // NS
    mesh = plsc.VectorSubcoreMesh(core_axis_name="c", subcore_axis_name="s")

    @functools.partial(
        pl.kernel,
        out_type=jax.ShapeDtypeStruct((2 * NP, acc_w), jnp.float32),
        mesh=mesh,
        scratch_types=[
            pltpu.VMEM((3, G, CH), jnp.int32),
            pltpu.VMEM((3, G, CH), jnp.int32),
            pltpu.VMEM((2, CH, width), jnp.float32),
            pltpu.VMEM((2, CH, width), jnp.float32),
            pltpu.VMEM((2, CH, acc_w), jnp.float32),
            pltpu.VMEM((width // 16, 16), jnp.float32),
            pltpu.VMEM_SHARED((NP, acc_w), jnp.float32),
            pltpu.SemaphoreType.DMA,
            pltpu.SemaphoreType.DMA,
            pltpu.SemaphoreType.DMA,
            pltpu.SemaphoreType.DMA,
            pltpu.SemaphoreType.DMA,
            pltpu.SemaphoreType.DMA,
            pltpu.SemaphoreType.DMA,
        ],
        compiler_params=_SC_PARAMS,
    )
    def fused_kernel(tl, tr, s3d, d3d, att, zeros, out,
                     sidx, didx, xj_v, xi_v, ov, att_v, acc,
                     gj0, gj1, gi0, gi1, ss0, ss1, slab_sem):
        gj = (gj0, gj1)
        gi = (gi0, gi1)
        ss = (ss0, ss1)
        c = lax.axis_index("c")
        s = lax.axis_index("s")
        wid = c * NS + s

        pltpu.sync_copy(att, att_v)
        r0 = s * rows_per_tile
        pltpu.sync_copy(zeros, acc.at[pl.ds(r0, rows_per_tile)])

        def start_slab(g, slot):
            pltpu.async_copy(s3d.at[wid, g], sidx.at[slot], slab_sem)
            pltpu.async_copy(d3d.at[wid, g], didx.at[slot], slab_sem)

        def wait_slab():
            pltpu.make_async_copy(s3d.at[wid, 0], sidx.at[0],
                                  slab_sem).wait()
            pltpu.make_async_copy(d3d.at[wid, 0], didx.at[0],
                                  slab_sem).wait()

        start_slab(0, 0)
        wait_slab()
        start_slab(1, 1)
        plsc.subcore_barrier()

        vr_per_h = n_vr // n_heads
        att_r = [att_v[r, :] for r in range(n_vr)]
        iota = lax.iota(jnp.int32, 16)
        onehot = [iota == h for h in range(n_heads)]
        def compute_chunk(b, slot, j):
            @plsc.parallel_loop(0, CH, 1, unroll=unroll)
            def edge_body(e):
                alpha = jnp.zeros((16,), jnp.float32)
                for h in range(n_heads):
                    acc_s = jnp.float32(0.0)
                    for v in range(vr_per_h):
                        r = h * vr_per_h + v
                        sl = pl.ds(r * 16, 16)
                        x = xi_v[b, e, sl] + xj_v[b, e, sl]
                        acc_s = acc_s + jnp.sum(
                            jnp.maximum(x, 0.2 * x) * att_r[r])
                    alpha = jnp.where(onehot[h],
                                      jax.lax.broadcast(acc_s, (16,)), alpha)
                w = jnp.exp(alpha)
                for h in range(n_heads):
                    wh = jax.lax.broadcast(w[h], (16,))
                    for v in range(vr_per_h):
                        r = h * vr_per_h + v
                        sl = pl.ds(r * 16, 16)
                        ov[b, e, sl] = xj_v[b, e, sl] * wh
                ov[b, e, pl.ds(width, 16)] = w

        def start_gathers(b, slot, j):
            pltpu.async_copy(tl.at[sidx.at[slot, j]], xj_v.at[b], gj[b])
            pltpu.async_copy(tr.at[didx.at[slot, j]], xi_v.at[b], gi[b])

        def wait_gathers(b, slot, j):
            pltpu.make_async_copy(tl.at[sidx.at[slot, j]], xj_v.at[b],
                                  gj[b]).wait()
            pltpu.make_async_copy(tr.at[didx.at[slot, j]], xi_v.at[b],
                                  gi[b]).wait()

        start_gathers(0, 0, 0)
        start_gathers(1, 0, 1)

        def group_body(g, carry):
            g3 = g % 3

            @pl.when(g + 1 < n_grp)
            def _():
                wait_slab()

            @pl.when(g + 2 < n_grp)
            def _():
                start_slab(g + 2, (g + 2) % 3)

            for j in range(G):
                b = j % 2
                k = g * G + j
                wait_gathers(b, g3, j)

                @pl.when(k >= 2)
                def _():
                    pltpu.make_async_copy(ov.at[b], acc.at[didx.at[g3, j]],
                                          ss[b]).wait()

                compute_chunk(b, g3, j)
                pltpu.async_copy(ov.at[b], acc.at[didx.at[g3, j]], ss[b],
                                 add=True)
                if j < G - 2:
                    start_gathers(b, g3, j + 2)
                else:
                    @pl.when(g + 1 < n_grp)
                    def _():
                        start_gathers(b, (g + 1) % 3, j + 2 - G)
            return carry

        lax.fori_loop(0, n_grp, group_body, 0)
        for b in range(2):
            pltpu.make_async_copy(ov.at[b], acc.at[didx.at[0, 0]],
                                  ss[b]).wait()
        plsc.subcore_barrier()
        pltpu.sync_copy(acc.at[pl.ds(r0, rows_per_tile)],
                        out.at[pl.ds(c * NP + r0, rows_per_tile)])

    return fused_kernel


def _project(x, wl, bl, wr, br):
    n, d = x.shape
    f = wl.shape[1]

    def body(x_ref, wl_ref, bl_ref, wr_ref, br_ref, xl_ref, xr_ref):
        xb = x_ref[...]
        xl_ref[...] = jnp.dot(xb, wl_ref[...],
                              preferred_element_type=jnp.float32) + bl_ref[...]
        xr_ref[...] = jnp.dot(xb, wr_ref[...],
                              preferred_element_type=jnp.float32) + br_ref[...]

    return pl.pallas_call(
        body,
        grid=(n // BN,),
        in_specs=[
            pl.BlockSpec((BN, d), lambda i: (i, 0)),
            pl.BlockSpec((d, f), lambda i: (0, 0)),
            pl.BlockSpec((1, f), lambda i: (0, 0)),
            pl.BlockSpec((d, f), lambda i: (0, 0)),
            pl.BlockSpec((1, f), lambda i: (0, 0)),
        ],
        out_specs=[pl.BlockSpec((BN, f), lambda i: (i, 0))] * 2,
        out_shape=[jax.ShapeDtypeStruct((n, f), jnp.float32)] * 2,
    )(x, wl, bl.reshape(1, -1), wr, br.reshape(1, -1))


def _mid_stage(acc0, acc1, xl1, xr1, a_mat, b_mat, bias1,
               wl2p, bl2p, wr2p, br2p, att2p):
    def body(a0_ref, a1_ref, xl_ref, xr_ref, a_ref, b_ref, b1_ref,
             wl2_ref, bl2_ref, wr2_ref, br2_ref, att2_ref,
             xl2_ref, xr2_ref, s2_ref):
        xl_ = xl_ref[...]
        xr_ = xr_ref[...]
        l = _lrelu(xl_ + xr_)
        wii = jnp.exp(jnp.dot(l, a_ref[...],
                              preferred_element_type=jnp.float32))
        t0 = a0_ref[...]
        t1 = a1_ref[...]
        num = (t0[:, :F1] + t1[:, :F1]
               + jnp.dot(wii, b_ref[...],
                         preferred_element_type=jnp.float32) * xl_)
        den = t0[:, F1:F1 + H1] + t1[:, F1:F1 + H1] + wii
        inv = 1.0 / (den + 1e-16)
        hin = num * jnp.dot(inv, b_ref[...],
                            preferred_element_type=jnp.float32) + b1_ref[...]
        h = jnp.where(hin > 0, hin, jnp.exp(jnp.minimum(hin, 0.0)) - 1.0)
        xl2 = jnp.dot(h, wl2_ref[...],
                      preferred_element_type=jnp.float32) + bl2_ref[...]
        xr2 = jnp.dot(h, wr2_ref[...],
                      preferred_element_type=jnp.float32) + br2_ref[...]
        l2 = _lrelu(xl2 + xr2)
        w2 = jnp.exp(jnp.dot(l2, att2_ref[...],
                             preferred_element_type=jnp.float32))
        z = jnp.zeros((xl2.shape[0], ACC2_W - C2P - 1), jnp.float32)
        xl2_ref[...] = xl2
        xr2_ref[...] = xr2
        s2_ref[...] = jnp.concatenate([w2 * xl2, w2, z], axis=1)

    nblk = NP // BN
    return pl.pallas_call(
        body,
        grid=(NP // BN,),
        in_specs=[
            pl.BlockSpec((BN, ACC1_W), lambda i: (i, 0)),
            pl.BlockSpec((BN, ACC1_W), lambda i, n=nblk: (i + n, 0)),
            pl.BlockSpec((BN, F1), lambda i: (i, 0)),
            pl.BlockSpec((BN, F1), lambda i: (i, 0)),
            pl.BlockSpec((F1, H1), lambda i: (0, 0)),
            pl.BlockSpec((H1, F1), lambda i: (0, 0)),
            pl.BlockSpec((1, F1), lambda i: (0, 0)),
            pl.BlockSpec((F1, C2P), lambda i: (0, 0)),
            pl.BlockSpec((1, C2P), lambda i: (0, 0)),
            pl.BlockSpec((F1, C2P), lambda i: (0, 0)),
            pl.BlockSpec((1, C2P), lambda i: (0, 0)),
            pl.BlockSpec((C2P, 1), lambda i: (0, 0)),
        ],
        out_specs=[
            pl.BlockSpec((BN, C2P), lambda i: (i, 0)),
            pl.BlockSpec((BN, C2P), lambda i: (i, 0)),
            pl.BlockSpec((BN, ACC2_W), lambda i: (i, 0)),
        ],
        out_shape=[
            jax.ShapeDtypeStruct((NP, C2P), jnp.float32),
            jax.ShapeDtypeStruct((NP, C2P), jnp.float32),
            jax.ShapeDtypeStruct((NP, ACC2_W), jnp.float32),
        ],
    )(acc0, acc1, xl1, xr1, a_mat, b_mat, bias1,
      wl2p, bl2p, wr2p, br2p, att2p)


def _final_stage(acc0, acc1, s2, bias2):
    def body(a0_ref, a1_ref, s2_ref, b2_ref, out_ref):
        t = a0_ref[...] + a1_ref[...] + s2_ref[...]
        den = t[:, C2P:C2P + 1]
        out_ref[...] = t[:, :C2] / (den + 1e-16) + b2_ref[...]

    nblk = NP // BN
    return pl.pallas_call(
        body,
        grid=(NP // BN,),
        in_specs=[
            pl.BlockSpec((BN, ACC2_W), lambda i: (i, 0)),
            pl.BlockSpec((BN, ACC2_W), lambda i, n=nblk: (i + n, 0)),
            pl.BlockSpec((BN, ACC2_W), lambda i: (i, 0)),
            pl.BlockSpec((1, C2), lambda i: (0, 0)),
        ],
        out_specs=pl.BlockSpec((BN, C2), lambda i: (i, 0)),
        out_shape=jax.ShapeDtypeStruct((NP, C2), jnp.float32),
    )(acc0, acc1, s2, bias2.reshape(1, -1))


CH1, G1 = 40, 10
CH2, G2 = 100, 10
_edge_pass1 = _make_fused_edge_pass(F1, ACC1_W, H1, unroll=2, ch=CH1, grp=G1)
_edge_pass2 = _make_fused_edge_pass(C2P, ACC2_W, 1, unroll=8, ch=CH2, grp=G2)


def kernel(x, edge_index, Wl1, bl1, Wr1, br1, att1, bias1,
           Wl2, bl2, Wr2, br2, att2, bias2):
    s3d1 = edge_index[0].reshape(NW, E // (NW * G1 * CH1), G1, CH1)
    d3d1 = edge_index[1].reshape(NW, E // (NW * G1 * CH1), G1, CH1)
    s3d2 = edge_index[0].reshape(NW, E // (NW * G2 * CH2), G2, CH2)
    d3d2 = edge_index[1].reshape(NW, E // (NW * G2 * CH2), G2, CH2)

    a1 = (att1[:, :, None] * jnp.eye(H1, dtype=jnp.float32)[:, None, :]
          ).reshape(F1, H1)
    bm = jnp.repeat(jnp.eye(H1, dtype=jnp.float32), C1, axis=1)
    att2p = jnp.zeros((C2P, 1), jnp.float32).at[:C2, 0].set(att2[0])
    att2v = att2p.reshape(3, 16)
    wl2p = jnp.pad(Wl2, ((0, 0), (0, C2P - C2)))
    wr2p = jnp.pad(Wr2, ((0, 0), (0, C2P - C2)))
    bl2p = jnp.pad(bl2, (0, C2P - C2)).reshape(1, -1)
    br2p = jnp.pad(br2, (0, C2P - C2)).reshape(1, -1)
    zeros1 = jnp.zeros((NP // NS, ACC1_W), jnp.float32)
    zeros2 = jnp.zeros((NP // NS, ACC2_W), jnp.float32)

    xp = jnp.pad(x, ((0, NP - N), (0, 0)))
    xl1, xr1 = _project(xp, Wl1, bl1, Wr1, br1)
    acc1 = _edge_pass1(xl1, xr1, s3d1, d3d1, att1, zeros1)
    xl2, xr2, s2 = _mid_stage(acc1, acc1, xl1, xr1, a1, bm,
                              bias1.reshape(1, -1), wl2p, bl2p, wr2p, br2p,
                              att2p)
    acc2 = _edge_pass2(xl2, xr2, s3d2, d3d2, att2v, zeros2)
    return _final_stage(acc2, acc2, s2, bias2)[:N]
